# baseline (speedup 1.0000x reference)
import functools

import jax
import jax.numpy as jnp
from jax import lax
from jax.experimental import pallas as pl
from jax.experimental.pallas import tpu as pltpu
from jax.experimental.pallas import tpu_sc as plsc

NC = 2
NS = 16
NW = NC * NS
LN = 16

RC = 80


def _bcast16(v, e):
    sc_scalar = lax.squeeze(lax.slice(v, (e,), (e + 1,)), (0,))
    return jnp.broadcast_to(sc_scalar, (LN,))


@functools.cache
def _sc_a(n, vocab, ed, e_edges):
    n_chunks = n // RC
    rr = -(-n_chunks // NW)
    deg_edges_per_sub = e_edges // NS
    deg_iters = deg_edges_per_sub // (5 * RC)
    npad = 10240
    mesh = plsc.VectorSubcoreMesh(core_axis_name="c", subcore_axis_name="s")

    @functools.partial(
        pl.kernel,
        out_type=(
            jax.ShapeDtypeStruct((n, ed), jnp.float32),
            jax.ShapeDtypeStruct((n, 128), jnp.float32),
        ),
        mesh=mesh,
        scratch_types=[
            pltpu.VMEM_SHARED((npad,), jnp.float32),
            pltpu.VMEM((640,), jnp.float32),
            pltpu.VMEM((5, RC), jnp.int32),
            pltpu.VMEM((5, RC), jnp.float32),
            pltpu.VMEM((RC,), jnp.int32),
            pltpu.VMEM((RC,), jnp.float32),
            pltpu.VMEM((RC, 128), jnp.float32),
            pltpu.SemaphoreType.DMA,
        ],
    )
    def body(ids_hbm, emb_hbm, dstr_hbm, wr_hbm, embed_out, dinv_out,
             acc, zbuf, didx2, wbuf2, idxb, dbuf, rows, sem):
        c = lax.axis_index("c")
        s = lax.axis_index("s")
        wid = s * NC + c

        def zb(i, _):
            zbuf[pl.ds(pl.multiple_of(i * LN, LN), LN)] = jnp.zeros(
                (LN,), jnp.float32)
            return 0
        lax.fori_loop(0, 640 // LN, zb, 0)
        pltpu.sync_copy(zbuf, acc.at[pl.ds(pl.multiple_of(s * 640, 8), 640)])
        plsc.subcore_barrier()

        def deg_step(t, _):
            base = s * deg_edges_per_sub + t * (5 * RC)
            for k in range(5):
                off = pl.multiple_of(base + k * RC, RC)
                pltpu.sync_copy(dstr_hbm.at[pl.ds(off, RC)], didx2.at[k])
                pltpu.sync_copy(wr_hbm.at[pl.ds(off, RC)], wbuf2.at[k])
            descs = [
                pltpu.async_copy(wbuf2.at[k], acc.at[didx2.at[k]], sem,
                                 add=True)
                for k in range(5)
            ]
            for d_ in descs:
                d_.wait()
            return 0
        lax.fori_loop(0, deg_iters, deg_step, 0)
        plsc.subcore_barrier()

        for k in range(rr):
            cid = wid + k * NW

            @pl.when(cid < n_chunks)
            def _():
                off = pl.multiple_of(cid * RC, RC)
                pltpu.sync_copy(acc.at[pl.ds(off, RC)], dbuf)
                for g in range(RC // LN):
                    d16 = dbuf[pl.ds(g * LN, LN)] + 1.0
                    for e in range(LN):
                        sc_val = _bcast16(d16, e)
                        r = g * LN + e
                        for j in range(128 // LN):
                            rows[r, pl.ds(j * LN, LN)] = sc_val
                pltpu.sync_copy(rows, dinv_out.at[pl.ds(off, RC)])

        for k in range(rr):
            cid = wid + k * NW

            @pl.when(cid < n_chunks)
            def _():
                off = pl.multiple_of(cid * RC, RC)
                pltpu.sync_copy(ids_hbm.at[pl.ds(off, RC)], idxb)
                pltpu.async_copy(emb_hbm.at[idxb], rows, sem).wait()
                pltpu.sync_copy(rows, embed_out.at[pl.ds(off, RC)])

    return body


@functools.cache
def _sc_b(n, e_edges):
    epw = e_edges // NW
    n_ch = epw // RC
    npad = 10240
    mesh = plsc.VectorSubcoreMesh(core_axis_name="c", subcore_axis_name="s")

    @functools.partial(
        pl.kernel,
        out_type=jax.ShapeDtypeStruct((NC, n, 128), jnp.float32),
        mesh=mesh,
        scratch_types=[
            pltpu.VMEM_SHARED((npad, 128), jnp.float32),
            pltpu.VMEM((3, RC), jnp.int32),
            pltpu.VMEM((3, RC), jnp.int32),
            pltpu.VMEM((3, RC), jnp.float32),
            pltpu.VMEM((3, RC, 128), jnp.float32),
            pltpu.SemaphoreType.DMA((3,)),
            pltpu.SemaphoreType.DMA((3,)),
            pltpu.SemaphoreType.DMA((3,)),
        ],
    )
    def body(hhat_hbm, src_hbm, dst_hbm, w_hbm, zeros_hbm, part_out,
             acc, sidx3, didx3, wbf3, rows3, isem, gsem, ssem):
        c = lax.axis_index("c")
        s = lax.axis_index("s")
        wid = s * NC + c

        pltpu.sync_copy(zeros_hbm, acc.at[pl.ds(pl.multiple_of(s * 640, 8),
                                                640), :])
        plsc.subcore_barrier()

        ebase = wid * epw

        def it(r, _):
            j0 = lax.rem(r, 3)
            j1 = lax.rem(r + 2, 3)
            j2 = lax.rem(r + 1, 3)

            @pl.when(r < n_ch)
            def _():
                @pl.when(r >= 3)
                def _():
                    pltpu.make_async_copy(
                        rows3.at[j0], acc.at[didx3.at[j0]],
                        ssem.at[j0]).wait()
                off = pl.multiple_of(ebase + r * RC, RC)
                pltpu.async_copy(src_hbm.at[pl.ds(off, RC)], sidx3.at[j0],
                                 isem.at[j0])
                pltpu.async_copy(dst_hbm.at[pl.ds(off, RC)], didx3.at[j0],
                                 isem.at[j0])
                pltpu.async_copy(w_hbm.at[pl.ds(off, RC)], wbf3.at[j0],
                                 isem.at[j0])

            @pl.when((r >= 1) & (r <= n_ch))
            def _():
                pltpu.make_async_copy(src_hbm.at[pl.ds(0, RC)],
                                      sidx3.at[j1], isem.at[j1]).wait()
                pltpu.make_async_copy(dst_hbm.at[pl.ds(0, RC)],
                                      didx3.at[j1], isem.at[j1]).wait()
                pltpu.make_async_copy(w_hbm.at[pl.ds(0, RC)],
                                      wbf3.at[j1], isem.at[j1]).wait()
                pltpu.async_copy(hhat_hbm.at[sidx3.at[j1]], rows3.at[j1],
                                 gsem.at[j1])

            @pl.when(r >= 2)
            def _():
                pltpu.make_async_copy(hhat_hbm.at[sidx3.at[j2]],
                                      rows3.at[j2], gsem.at[j2]).wait()

                def gbody(g, _):
                    wv = wbf3[j2, pl.ds(g * LN, LN)]
                    for e in range(LN):
                        sc_val = _bcast16(wv, e)
                        rr = g * LN + e
                        for j in range(128 // LN):
                            rows3[j2, rr, pl.ds(j * LN, LN)] = (
                                rows3[j2, rr, pl.ds(j * LN, LN)] * sc_val)
                    return 0

                lax.fori_loop(0, RC // LN, gbody, 0)
                pltpu.async_copy(rows3.at[j2], acc.at[didx3.at[j2]],
                                 ssem.at[j2], add=True)

            return 0

        lax.fori_loop(0, n_ch + 2, it, 0)
        for t in range(3):
            jt = (n_ch - 1 - t) % 3
            pltpu.make_async_copy(rows3.at[jt], acc.at[didx3.at[jt]],
                                  ssem.at[jt]).wait()
        plsc.subcore_barrier()

        @pl.when(s < NS - 1)
        def _():
            off = pl.multiple_of(s * 640, 8)
            pltpu.sync_copy(acc.at[pl.ds(off, 640), :],
                            part_out.at[c, pl.ds(off, 640), :])

        @pl.when(s == NS - 1)
        def _():
            pltpu.sync_copy(acc.at[pl.ds(9600, 400), :],
                            part_out.at[c, pl.ds(9600, 400), :])

    return body


@functools.cache
def _tc1(n, d, ed, h, blk):
    grid = (n // blk,)

    def body(x_ref, e_ref, dg_ref, wx_ref, we_ref, ht_ref, hh_ref, dv_ref):
        dv = lax.rsqrt(dg_ref[...])
        ht = (jnp.dot(x_ref[...], wx_ref[...],
                      preferred_element_type=jnp.float32)
              + jnp.dot(e_ref[...], we_ref[...],
                        preferred_element_type=jnp.float32))
        ht_ref[...] = ht
        hh_ref[...] = ht * dv
        dv_ref[...] = dv

    return pl.pallas_call(
        body,
        grid=grid,
        in_specs=[
            pl.BlockSpec((blk, d), lambda i: (i, 0)),
            pl.BlockSpec((blk, ed), lambda i: (i, 0)),
            pl.BlockSpec((blk, 128), lambda i: (i, 0)),
            pl.BlockSpec((d, h), lambda i: (0, 0)),
            pl.BlockSpec((ed, h), lambda i: (0, 0)),
        ],
        out_specs=[
            pl.BlockSpec((blk, h), lambda i: (i, 0)),
            pl.BlockSpec((blk, h), lambda i: (i, 0)),
            pl.BlockSpec((blk, 128), lambda i: (i, 0)),
        ],
        out_shape=[
            jax.ShapeDtypeStruct((n, h), jnp.float32),
            jax.ShapeDtypeStruct((n, h), jnp.float32),
            jax.ShapeDtypeStruct((n, 128), jnp.float32),
        ],
    )


@functools.cache
def _tc2(n, h, blk):
    grid = (n // blk,)

    def body(p_ref, dv_ref, ht1_ref, b1_ref, w2_ref, ht2_ref, hh2_ref):
        dv = dv_ref[...]
        agg = ((p_ref[0] + p_ref[1]) * dv
               + ht1_ref[...] * dv * dv + b1_ref[...])
        hrelu = jnp.maximum(agg, 0.0)
        ht2 = jnp.dot(hrelu, w2_ref[...], preferred_element_type=jnp.float32)
        ht2_ref[...] = ht2
        hh2_ref[...] = ht2 * dv

    return pl.pallas_call(
        body,
        grid=grid,
        in_specs=[
            pl.BlockSpec((NC, blk, 128), lambda i: (0, i, 0)),
            pl.BlockSpec((blk, 128), lambda i: (i, 0)),
            pl.BlockSpec((blk, 128), lambda i: (i, 0)),
            pl.BlockSpec((1, 128), lambda i: (0, 0)),
            pl.BlockSpec((h, h), lambda i: (0, 0)),
        ],
        out_specs=[
            pl.BlockSpec((blk, h), lambda i: (i, 0)),
            pl.BlockSpec((blk, h), lambda i: (i, 0)),
        ],
        out_shape=[
            jax.ShapeDtypeStruct((n, h), jnp.float32),
            jax.ShapeDtypeStruct((n, h), jnp.float32),
        ],
    )


@functools.cache
def _tc3(n, h, out_dim, g_graphs, blk):
    nblk = n // blk
    grid = (nblk,)

    def body(p_ref, dv_ref, ht2_ref, b2_ref, b_ref, wfc_ref, bfc_ref,
             out_ref, acc_ref, cnt_ref):
        i = pl.program_id(0)
        dv = dv_ref[...]
        agg = ((p_ref[0] + p_ref[1]) * dv
               + ht2_ref[...] * dv * dv + b2_ref[...])
        hrelu = jnp.maximum(agg, 0.0)
        bb = b_ref[0]
        gi = lax.broadcasted_iota(jnp.int32, (g_graphs, blk), 0)
        mt = (jnp.broadcast_to(bb, (g_graphs, blk)) == gi
              ).astype(jnp.float32)
        s_blk = jnp.dot(mt, hrelu, preferred_element_type=jnp.float32)
        c_blk = jnp.dot(mt, jnp.ones((blk, h), jnp.float32),
                        preferred_element_type=jnp.float32)

        @pl.when(i == 0)
        def _():
            acc_ref[...] = s_blk
            cnt_ref[...] = c_blk

        @pl.when(i > 0)
        def _():
            acc_ref[...] += s_blk
            cnt_ref[...] += c_blk

        @pl.when(i == nblk - 1)
        def _():
            pooled = acc_ref[...] / jnp.maximum(cnt_ref[...], 1.0)
            out_ref[...] = (jnp.dot(pooled, wfc_ref[...],
                                    preferred_element_type=jnp.float32)
                            + bfc_ref[...])

    return pl.pallas_call(
        body,
        grid=grid,
        in_specs=[
            pl.BlockSpec((NC, blk, 128), lambda i: (0, i, 0)),
            pl.BlockSpec((blk, 128), lambda i: (i, 0)),
            pl.BlockSpec((blk, 128), lambda i: (i, 0)),
            pl.BlockSpec((1, 128), lambda i: (0, 0)),
            pl.BlockSpec((1, 1, blk), lambda i: (i, 0, 0)),
            pl.BlockSpec((h, out_dim), lambda i: (0, 0)),
            pl.BlockSpec((1, out_dim), lambda i: (0, 0)),
        ],
        out_specs=pl.BlockSpec((g_graphs, out_dim), lambda i: (0, 0)),
        out_shape=jax.ShapeDtypeStruct((g_graphs, out_dim), jnp.float32),
        scratch_shapes=[
            pltpu.VMEM((g_graphs, h), jnp.float32),
            pltpu.VMEM((g_graphs, h), jnp.float32),
        ],
    )


def kernel(x, edge_index, edge_attr, batch, node_ids, emb_table,
           W1, b1, W2, b2, Wfc, bfc):
    n, d = x.shape
    e_edges = edge_attr.shape[0]
    ed = emb_table.shape[1]
    h = W2.shape[0]
    out_dim = Wfc.shape[1]
    g_graphs = 64
    blk = 1000

    src = edge_index[0]
    dst = edge_index[1]
    zeros = jnp.zeros((640, 128), jnp.float32)

    embed, deg2d = _sc_a(n, emb_table.shape[0], ed, e_edges)(
        node_ids, emb_table, dst, edge_attr)

    ht1, hhat1, dinv2d = _tc1(n, d, ed, h, blk)(
        x, embed, deg2d, W1[:d], W1[d:])

    part1 = _sc_b(n, e_edges)(hhat1, src, dst, edge_attr, zeros)

    ht2, hhat2 = _tc2(n, h, blk)(
        part1, dinv2d, ht1, b1.reshape(1, h), W2)

    part2 = _sc_b(n, e_edges)(hhat2, src, dst, edge_attr, zeros)

    out = _tc3(n, h, out_dim, g_graphs, blk)(
        part2, dinv2d, ht2, b2.reshape(1, h),
        batch.reshape(n // blk, 1, blk), Wfc, bfc.reshape(1, out_dim))
    return out


# speedup vs baseline: 1.9945x; 1.9945x over previous
import functools

import jax
import jax.numpy as jnp
from jax import lax
from jax.experimental import pallas as pl
from jax.experimental.pallas import tpu as pltpu
from jax.experimental.pallas import tpu_sc as plsc

NC = 2
NS = 16
NW = NC * NS
LN = 16

RC = 80


def _bcast16(v, e):
    sc_scalar = lax.squeeze(lax.slice(v, (e,), (e + 1,)), (0,))
    return jnp.broadcast_to(sc_scalar, (LN,))


@functools.cache
def _sc_a(n, vocab, ed, e_edges):
    n_chunks = n // RC
    rr = -(-n_chunks // NW)
    deg_edges_per_sub = e_edges // NS
    deg_iters = deg_edges_per_sub // (5 * RC)
    npad = 10240
    mesh = plsc.VectorSubcoreMesh(core_axis_name="c", subcore_axis_name="s")

    @functools.partial(
        pl.kernel,
        out_type=(
            jax.ShapeDtypeStruct((n, ed), jnp.float32),
            jax.ShapeDtypeStruct((n, 128), jnp.float32),
        ),
        mesh=mesh,
        scratch_types=[
            pltpu.VMEM_SHARED((npad,), jnp.float32),
            pltpu.VMEM((640,), jnp.float32),
            pltpu.VMEM((5, RC), jnp.int32),
            pltpu.VMEM((5, RC), jnp.float32),
            pltpu.VMEM((RC,), jnp.int32),
            pltpu.VMEM((RC,), jnp.float32),
            pltpu.VMEM((RC, 128), jnp.float32),
            pltpu.SemaphoreType.DMA,
        ],
    )
    def body(ids_hbm, emb_hbm, dstr_hbm, wr_hbm, embed_out, dinv_out,
             acc, zbuf, didx2, wbuf2, idxb, dbuf, rows, sem):
        c = lax.axis_index("c")
        s = lax.axis_index("s")
        wid = s * NC + c

        def zb(i, _):
            zbuf[pl.ds(pl.multiple_of(i * LN, LN), LN)] = jnp.zeros(
                (LN,), jnp.float32)
            return 0
        lax.fori_loop(0, 640 // LN, zb, 0)
        pltpu.sync_copy(zbuf, acc.at[pl.ds(pl.multiple_of(s * 640, 8), 640)])
        plsc.subcore_barrier()

        def deg_step(t, _):
            base = s * deg_edges_per_sub + t * (5 * RC)
            for k in range(5):
                off = pl.multiple_of(base + k * RC, RC)
                pltpu.sync_copy(dstr_hbm.at[pl.ds(off, RC)], didx2.at[k])
                pltpu.sync_copy(wr_hbm.at[pl.ds(off, RC)], wbuf2.at[k])
            descs = [
                pltpu.async_copy(wbuf2.at[k], acc.at[didx2.at[k]], sem,
                                 add=True)
                for k in range(5)
            ]
            for d_ in descs:
                d_.wait()
            return 0
        lax.fori_loop(0, deg_iters, deg_step, 0)
        plsc.subcore_barrier()

        for k in range(rr):
            cid = wid + k * NW

            @pl.when(cid < n_chunks)
            def _():
                off = pl.multiple_of(cid * RC, RC)
                pltpu.sync_copy(acc.at[pl.ds(off, RC)], dbuf)
                for g in range(RC // LN):
                    d16 = dbuf[pl.ds(g * LN, LN)] + 1.0
                    for e in range(LN):
                        sc_val = _bcast16(d16, e)
                        r = g * LN + e
                        for j in range(128 // LN):
                            rows[r, pl.ds(j * LN, LN)] = sc_val
                pltpu.sync_copy(rows, dinv_out.at[pl.ds(off, RC)])

        for k in range(rr):
            cid = wid + k * NW

            @pl.when(cid < n_chunks)
            def _():
                off = pl.multiple_of(cid * RC, RC)
                pltpu.sync_copy(ids_hbm.at[pl.ds(off, RC)], idxb)
                pltpu.async_copy(emb_hbm.at[idxb], rows, sem).wait()
                pltpu.sync_copy(rows, embed_out.at[pl.ds(off, RC)])

    return body


@functools.cache
def _sc_b(n, e_edges):
    epw = e_edges // NW
    n_ch = epw // RC
    npad = 10240
    mesh = plsc.VectorSubcoreMesh(core_axis_name="c", subcore_axis_name="s")

    @functools.partial(
        pl.kernel,
        out_type=jax.ShapeDtypeStruct((NC, n, 128), jnp.float32),
        mesh=mesh,
        scratch_types=[
            pltpu.VMEM_SHARED((npad, 128), jnp.float32),
            pltpu.VMEM((3, RC), jnp.int32),
            pltpu.VMEM((3, RC), jnp.int32),
            pltpu.VMEM((3, RC), jnp.float32),
            pltpu.VMEM((3, RC, 128), jnp.float32),
            pltpu.SemaphoreType.DMA((3,)),
            pltpu.SemaphoreType.DMA((3,)),
            pltpu.SemaphoreType.DMA((3,)),
        ],
    )
    def body(hhat_hbm, src_hbm, dst_hbm, w_hbm, zeros_hbm, part_out,
             acc, sidx3, didx3, wbf3, rows3, isem, gsem, ssem):
        c = lax.axis_index("c")
        s = lax.axis_index("s")
        wid = s * NC + c

        pltpu.sync_copy(zeros_hbm, acc.at[pl.ds(pl.multiple_of(s * 640, 8),
                                                640), :])
        plsc.subcore_barrier()

        ebase = wid * epw

        def it(r, _):
            j0 = lax.rem(r, 3)
            j1 = lax.rem(r + 2, 3)
            j2 = lax.rem(r + 1, 3)

            @pl.when(r < n_ch)
            def _():
                @pl.when(r >= 3)
                def _():
                    pltpu.make_async_copy(
                        rows3.at[j0], acc.at[didx3.at[j0]],
                        ssem.at[j0]).wait()
                off = pl.multiple_of(ebase + r * RC, RC)
                pltpu.async_copy(src_hbm.at[pl.ds(off, RC)], sidx3.at[j0],
                                 isem.at[j0])
                pltpu.async_copy(dst_hbm.at[pl.ds(off, RC)], didx3.at[j0],
                                 isem.at[j0])
                pltpu.async_copy(w_hbm.at[pl.ds(off, RC)], wbf3.at[j0],
                                 isem.at[j0])

            @pl.when((r >= 1) & (r <= n_ch))
            def _():
                pltpu.make_async_copy(src_hbm.at[pl.ds(0, RC)],
                                      sidx3.at[j1], isem.at[j1]).wait()
                pltpu.make_async_copy(dst_hbm.at[pl.ds(0, RC)],
                                      didx3.at[j1], isem.at[j1]).wait()
                pltpu.make_async_copy(w_hbm.at[pl.ds(0, RC)],
                                      wbf3.at[j1], isem.at[j1]).wait()
                pltpu.async_copy(hhat_hbm.at[sidx3.at[j1]], rows3.at[j1],
                                 gsem.at[j1])

            @pl.when(r >= 2)
            def _():
                pltpu.make_async_copy(hhat_hbm.at[sidx3.at[j2]],
                                      rows3.at[j2], gsem.at[j2]).wait()
                rview = rows3.at[j2]
                wview = wbf3.at[j2]
                for g in range(RC // LN):
                    wv = wview[pl.ds(g * LN, LN)]
                    for e in range(LN):
                        sc_val = _bcast16(wv, e)
                        rr = g * LN + e
                        for j in range(128 // LN):
                            rview[rr, pl.ds(j * LN, LN)] = (
                                rview[rr, pl.ds(j * LN, LN)] * sc_val)
                pltpu.async_copy(rows3.at[j2], acc.at[didx3.at[j2]],
                                 ssem.at[j2], add=True)

            return 0

        lax.fori_loop(0, n_ch + 2, it, 0)
        for t in range(3):
            jt = (n_ch - 1 - t) % 3
            pltpu.make_async_copy(rows3.at[jt], acc.at[didx3.at[jt]],
                                  ssem.at[jt]).wait()
        plsc.subcore_barrier()

        @pl.when(s < NS - 1)
        def _():
            off = pl.multiple_of(s * 640, 8)
            pltpu.sync_copy(acc.at[pl.ds(off, 640), :],
                            part_out.at[c, pl.ds(off, 640), :])

        @pl.when(s == NS - 1)
        def _():
            pltpu.sync_copy(acc.at[pl.ds(9600, 400), :],
                            part_out.at[c, pl.ds(9600, 400), :])

    return body


@functools.cache
def _tc1(n, d, ed, h, blk):
    grid = (n // blk,)

    def body(x_ref, e_ref, dg_ref, wx_ref, we_ref, ht_ref, hh_ref, dv_ref):
        dv = lax.rsqrt(dg_ref[...])
        ht = (jnp.dot(x_ref[...], wx_ref[...],
                      preferred_element_type=jnp.float32)
              + jnp.dot(e_ref[...], we_ref[...],
                        preferred_element_type=jnp.float32))
        ht_ref[...] = ht
        hh_ref[...] = ht * dv
        dv_ref[...] = dv

    return pl.pallas_call(
        body,
        grid=grid,
        in_specs=[
            pl.BlockSpec((blk, d), lambda i: (i, 0)),
            pl.BlockSpec((blk, ed), lambda i: (i, 0)),
            pl.BlockSpec((blk, 128), lambda i: (i, 0)),
            pl.BlockSpec((d, h), lambda i: (0, 0)),
            pl.BlockSpec((ed, h), lambda i: (0, 0)),
        ],
        out_specs=[
            pl.BlockSpec((blk, h), lambda i: (i, 0)),
            pl.BlockSpec((blk, h), lambda i: (i, 0)),
            pl.BlockSpec((blk, 128), lambda i: (i, 0)),
        ],
        out_shape=[
            jax.ShapeDtypeStruct((n, h), jnp.float32),
            jax.ShapeDtypeStruct((n, h), jnp.float32),
            jax.ShapeDtypeStruct((n, 128), jnp.float32),
        ],
    )


@functools.cache
def _tc2(n, h, blk):
    grid = (n // blk,)

    def body(p_ref, dv_ref, ht1_ref, b1_ref, w2_ref, ht2_ref, hh2_ref):
        dv = dv_ref[...]
        agg = ((p_ref[0] + p_ref[1]) * dv
               + ht1_ref[...] * dv * dv + b1_ref[...])
        hrelu = jnp.maximum(agg, 0.0)
        ht2 = jnp.dot(hrelu, w2_ref[...], preferred_element_type=jnp.float32)
        ht2_ref[...] = ht2
        hh2_ref[...] = ht2 * dv

    return pl.pallas_call(
        body,
        grid=grid,
        in_specs=[
            pl.BlockSpec((NC, blk, 128), lambda i: (0, i, 0)),
            pl.BlockSpec((blk, 128), lambda i: (i, 0)),
            pl.BlockSpec((blk, 128), lambda i: (i, 0)),
            pl.BlockSpec((1, 128), lambda i: (0, 0)),
            pl.BlockSpec((h, h), lambda i: (0, 0)),
        ],
        out_specs=[
            pl.BlockSpec((blk, h), lambda i: (i, 0)),
            pl.BlockSpec((blk, h), lambda i: (i, 0)),
        ],
        out_shape=[
            jax.ShapeDtypeStruct((n, h), jnp.float32),
            jax.ShapeDtypeStruct((n, h), jnp.float32),
        ],
    )


@functools.cache
def _tc3(n, h, out_dim, g_graphs, blk):
    nblk = n // blk
    grid = (nblk,)

    def body(p_ref, dv_ref, ht2_ref, b2_ref, b_ref, wfc_ref, bfc_ref,
             out_ref, acc_ref, cnt_ref):
        i = pl.program_id(0)
        dv = dv_ref[...]
        agg = ((p_ref[0] + p_ref[1]) * dv
               + ht2_ref[...] * dv * dv + b2_ref[...])
        hrelu = jnp.maximum(agg, 0.0)
        bb = b_ref[0]
        gi = lax.broadcasted_iota(jnp.int32, (g_graphs, blk), 0)
        mt = (jnp.broadcast_to(bb, (g_graphs, blk)) == gi
              ).astype(jnp.float32)
        s_blk = jnp.dot(mt, hrelu, preferred_element_type=jnp.float32)
        c_blk = jnp.dot(mt, jnp.ones((blk, h), jnp.float32),
                        preferred_element_type=jnp.float32)

        @pl.when(i == 0)
        def _():
            acc_ref[...] = s_blk
            cnt_ref[...] = c_blk

        @pl.when(i > 0)
        def _():
            acc_ref[...] += s_blk
            cnt_ref[...] += c_blk

        @pl.when(i == nblk - 1)
        def _():
            pooled = acc_ref[...] / jnp.maximum(cnt_ref[...], 1.0)
            out_ref[...] = (jnp.dot(pooled, wfc_ref[...],
                                    preferred_element_type=jnp.float32)
                            + bfc_ref[...])

    return pl.pallas_call(
        body,
        grid=grid,
        in_specs=[
            pl.BlockSpec((NC, blk, 128), lambda i: (0, i, 0)),
            pl.BlockSpec((blk, 128), lambda i: (i, 0)),
            pl.BlockSpec((blk, 128), lambda i: (i, 0)),
            pl.BlockSpec((1, 128), lambda i: (0, 0)),
            pl.BlockSpec((1, 1, blk), lambda i: (i, 0, 0)),
            pl.BlockSpec((h, out_dim), lambda i: (0, 0)),
            pl.BlockSpec((1, out_dim), lambda i: (0, 0)),
        ],
        out_specs=pl.BlockSpec((g_graphs, out_dim), lambda i: (0, 0)),
        out_shape=jax.ShapeDtypeStruct((g_graphs, out_dim), jnp.float32),
        scratch_shapes=[
            pltpu.VMEM((g_graphs, h), jnp.float32),
            pltpu.VMEM((g_graphs, h), jnp.float32),
        ],
    )


def kernel(x, edge_index, edge_attr, batch, node_ids, emb_table,
           W1, b1, W2, b2, Wfc, bfc):
    n, d = x.shape
    e_edges = edge_attr.shape[0]
    ed = emb_table.shape[1]
    h = W2.shape[0]
    out_dim = Wfc.shape[1]
    g_graphs = 64
    blk = 1000

    src = edge_index[0]
    dst = edge_index[1]
    zeros = jnp.zeros((640, 128), jnp.float32)

    embed, deg2d = _sc_a(n, emb_table.shape[0], ed, e_edges)(
        node_ids, emb_table, dst, edge_attr)

    ht1, hhat1, dinv2d = _tc1(n, d, ed, h, blk)(
        x, embed, deg2d, W1[:d], W1[d:])

    part1 = _sc_b(n, e_edges)(hhat1, src, dst, edge_attr, zeros)

    ht2, hhat2 = _tc2(n, h, blk)(
        part1, dinv2d, ht1, b1.reshape(1, h), W2)

    part2 = _sc_b(n, e_edges)(hhat2, src, dst, edge_attr, zeros)

    out = _tc3(n, h, out_dim, g_graphs, blk)(
        part2, dinv2d, ht2, b2.reshape(1, h),
        batch.reshape(n // blk, 1, blk), Wfc, bfc.reshape(1, out_dim))
    return out


# speedup vs baseline: 2.9616x; 1.4849x over previous
import functools

import jax
import jax.numpy as jnp
from jax import lax
from jax.experimental import pallas as pl
from jax.experimental.pallas import tpu as pltpu
from jax.experimental.pallas import tpu_sc as plsc

NC = 2
NS = 16
NW = NC * NS
LN = 16

RC = 80


def _bcast16(v, e):
    sc_scalar = lax.squeeze(lax.slice(v, (e,), (e + 1,)), (0,))
    return jnp.broadcast_to(sc_scalar, (LN,))


@functools.cache
def _sc_a(n, vocab, ed, e_edges):
    n_chunks = n // RC
    rr = -(-n_chunks // NW)
    ng = e_edges // RC // 8
    deg_rounds = -(-ng // NS)
    npad = 10240
    mesh = plsc.VectorSubcoreMesh(core_axis_name="c", subcore_axis_name="s")

    @functools.partial(
        pl.kernel,
        out_type=(
            jax.ShapeDtypeStruct((n, ed), jnp.float32),
            jax.ShapeDtypeStruct((n, 128), jnp.float32),
        ),
        mesh=mesh,
        scratch_types=[
            pltpu.VMEM_SHARED((npad,), jnp.float32),
            pltpu.VMEM((640,), jnp.float32),
            pltpu.VMEM((2, 8, RC), jnp.int32),
            pltpu.VMEM((2, 8, RC), jnp.float32),
            pltpu.VMEM((RC,), jnp.int32),
            pltpu.VMEM((RC,), jnp.float32),
            pltpu.VMEM((RC, 128), jnp.float32),
            pltpu.SemaphoreType.DMA,
            pltpu.SemaphoreType.DMA((2,)),
            pltpu.SemaphoreType.DMA((2,)),
        ],
    )
    def body(ids_hbm, emb_hbm, dstr_hbm, wr_hbm, embed_out, dinv_out,
             acc, zbuf, didx2, wbuf2, idxb, dbuf, rows, sem, isem, ssem):
        c = lax.axis_index("c")
        s = lax.axis_index("s")
        wid = s * NC + c

        def zb(i, _):
            zbuf[pl.ds(pl.multiple_of(i * LN, LN), LN)] = jnp.zeros(
                (LN,), jnp.float32)
            return 0
        lax.fori_loop(0, 640 // LN, zb, 0)
        pltpu.sync_copy(zbuf, acc.at[pl.ds(pl.multiple_of(s * 640, 8), 640)])
        plsc.subcore_barrier()

        def deg_load(t, slot):
            g = s + NS * t

            @pl.when(g < ng)
            def _():
                roff = pl.multiple_of(g * 8, 8)
                pltpu.async_copy(dstr_hbm.at[pl.ds(roff, 8), :],
                                 didx2.at[slot], isem.at[slot])
                pltpu.async_copy(wr_hbm.at[pl.ds(roff, 8), :],
                                 wbuf2.at[slot], isem.at[slot])

        def deg_wait_loads(slot):
            pltpu.make_async_copy(dstr_hbm.at[pl.ds(0, 8), :],
                                  didx2.at[slot], isem.at[slot]).wait()
            pltpu.make_async_copy(wr_hbm.at[pl.ds(0, 8), :],
                                  wbuf2.at[slot], isem.at[slot]).wait()

        def deg_drain_streams(t, slot):
            g = s + NS * t

            @pl.when((t >= 0) & (g < ng))
            def _():
                for k in range(8):
                    pltpu.make_async_copy(wbuf2.at[slot, k],
                                          acc.at[didx2.at[slot, k]],
                                          ssem.at[slot]).wait()

        deg_load(0, 0)

        def deg_step(t, _):
            p = lax.rem(t, 2)
            q = lax.rem(t + 1, 2)
            g = s + NS * t

            @pl.when(g < ng)
            def _():
                deg_wait_loads(p)
                for k in range(8):
                    pltpu.async_copy(wbuf2.at[p, k], acc.at[didx2.at[p, k]],
                                     ssem.at[p], add=True)
            deg_drain_streams(t - 1, q)
            deg_load(t + 1, q)
            return 0

        lax.fori_loop(0, deg_rounds, deg_step, 0)
        deg_drain_streams(deg_rounds - 1, (deg_rounds - 1) % 2)
        plsc.subcore_barrier()

        for k in range(rr):
            cid = wid + k * NW

            @pl.when(cid < n_chunks)
            def _():
                off = pl.multiple_of(cid * RC, RC)
                pltpu.sync_copy(acc.at[pl.ds(off, RC)], dbuf)
                for g in range(RC // LN):
                    d16 = dbuf[pl.ds(g * LN, LN)] + 1.0
                    for e in range(LN):
                        sc_val = _bcast16(d16, e)
                        r = g * LN + e
                        for j in range(128 // LN):
                            rows[r, pl.ds(j * LN, LN)] = sc_val
                pltpu.sync_copy(rows, dinv_out.at[pl.ds(off, RC)])

        for k in range(rr):
            cid = wid + k * NW

            @pl.when(cid < n_chunks)
            def _():
                off = pl.multiple_of(cid * RC, RC)
                pltpu.sync_copy(ids_hbm.at[pl.ds(off, RC)], idxb)
                pltpu.async_copy(emb_hbm.at[idxb], rows, sem).wait()
                pltpu.sync_copy(rows, embed_out.at[pl.ds(off, RC)])

    return body


@functools.cache
def _sc_b(n, e_edges):
    epw = e_edges // NW
    n_ch = epw // RC
    npad = 10240
    mesh = plsc.VectorSubcoreMesh(core_axis_name="c", subcore_axis_name="s")

    @functools.partial(
        pl.kernel,
        out_type=jax.ShapeDtypeStruct((NC, n, 128), jnp.float32),
        mesh=mesh,
        scratch_types=[
            pltpu.VMEM_SHARED((npad, 128), jnp.float32),
            pltpu.VMEM((3, RC), jnp.int32),
            pltpu.VMEM((3, RC), jnp.int32),
            pltpu.VMEM((3, RC), jnp.float32),
            pltpu.VMEM((3, RC, 128), jnp.float32),
            pltpu.SemaphoreType.DMA((3,)),
            pltpu.SemaphoreType.DMA((3,)),
            pltpu.SemaphoreType.DMA((3,)),
        ],
    )
    def body(hhat_hbm, src_hbm, dst_hbm, w_hbm, zeros_hbm, part_out,
             acc, sidx3, didx3, wbf3, rows3, isem, gsem, ssem):
        c = lax.axis_index("c")
        s = lax.axis_index("s")
        wid = s * NC + c

        pltpu.sync_copy(zeros_hbm, acc.at[pl.ds(pl.multiple_of(s * 640, 8),
                                                640), :])
        plsc.subcore_barrier()

        ebase = wid * epw

        def it(r, _):
            j0 = lax.rem(r, 3)
            j1 = lax.rem(r + 2, 3)
            j2 = lax.rem(r + 1, 3)

            @pl.when(r < n_ch)
            def _():
                @pl.when(r >= 3)
                def _():
                    pltpu.make_async_copy(
                        rows3.at[j0], acc.at[didx3.at[j0]],
                        ssem.at[j0]).wait()
                off = pl.multiple_of(ebase + r * RC, RC)
                pltpu.async_copy(src_hbm.at[pl.ds(off, RC)], sidx3.at[j0],
                                 isem.at[j0])
                pltpu.async_copy(dst_hbm.at[pl.ds(off, RC)], didx3.at[j0],
                                 isem.at[j0])
                pltpu.async_copy(w_hbm.at[pl.ds(off, RC)], wbf3.at[j0],
                                 isem.at[j0])

            @pl.when((r >= 1) & (r <= n_ch))
            def _():
                pltpu.make_async_copy(src_hbm.at[pl.ds(0, RC)],
                                      sidx3.at[j1], isem.at[j1]).wait()
                pltpu.make_async_copy(dst_hbm.at[pl.ds(0, RC)],
                                      didx3.at[j1], isem.at[j1]).wait()
                pltpu.make_async_copy(w_hbm.at[pl.ds(0, RC)],
                                      wbf3.at[j1], isem.at[j1]).wait()
                pltpu.async_copy(hhat_hbm.at[sidx3.at[j1]], rows3.at[j1],
                                 gsem.at[j1])

            @pl.when(r >= 2)
            def _():
                pltpu.make_async_copy(hhat_hbm.at[sidx3.at[j2]],
                                      rows3.at[j2], gsem.at[j2]).wait()
                rview = rows3.at[j2]
                wview = wbf3.at[j2]
                for g in range(RC // LN):
                    wv = wview[pl.ds(g * LN, LN)]
                    for e in range(LN):
                        sc_val = _bcast16(wv, e)
                        rr = g * LN + e
                        for j in range(128 // LN):
                            rview[rr, pl.ds(j * LN, LN)] = (
                                rview[rr, pl.ds(j * LN, LN)] * sc_val)
                pltpu.async_copy(rows3.at[j2], acc.at[didx3.at[j2]],
                                 ssem.at[j2], add=True)

            return 0

        lax.fori_loop(0, n_ch + 2, it, 0)
        for t in range(3):
            jt = (n_ch - 1 - t) % 3
            pltpu.make_async_copy(rows3.at[jt], acc.at[didx3.at[jt]],
                                  ssem.at[jt]).wait()
        plsc.subcore_barrier()

        @pl.when(s < NS - 1)
        def _():
            off = pl.multiple_of(s * 640, 8)
            pltpu.sync_copy(acc.at[pl.ds(off, 640), :],
                            part_out.at[c, pl.ds(off, 640), :])

        @pl.when(s == NS - 1)
        def _():
            pltpu.sync_copy(acc.at[pl.ds(9600, 400), :],
                            part_out.at[c, pl.ds(9600, 400), :])

    return body


@functools.cache
def _tc1(n, d, ed, h, blk):
    grid = (n // blk,)

    def body(x_ref, e_ref, dg_ref, wx_ref, we_ref, ht_ref, hh_ref, dv_ref):
        dv = lax.rsqrt(dg_ref[...])
        ht = (jnp.dot(x_ref[...], wx_ref[...],
                      preferred_element_type=jnp.float32)
              + jnp.dot(e_ref[...], we_ref[...],
                        preferred_element_type=jnp.float32))
        ht_ref[...] = ht
        hh_ref[...] = ht * dv
        dv_ref[...] = dv

    return pl.pallas_call(
        body,
        grid=grid,
        in_specs=[
            pl.BlockSpec((blk, d), lambda i: (i, 0)),
            pl.BlockSpec((blk, ed), lambda i: (i, 0)),
            pl.BlockSpec((blk, 128), lambda i: (i, 0)),
            pl.BlockSpec((d, h), lambda i: (0, 0)),
            pl.BlockSpec((ed, h), lambda i: (0, 0)),
        ],
        out_specs=[
            pl.BlockSpec((blk, h), lambda i: (i, 0)),
            pl.BlockSpec((blk, h), lambda i: (i, 0)),
            pl.BlockSpec((blk, 128), lambda i: (i, 0)),
        ],
        out_shape=[
            jax.ShapeDtypeStruct((n, h), jnp.float32),
            jax.ShapeDtypeStruct((n, h), jnp.float32),
            jax.ShapeDtypeStruct((n, 128), jnp.float32),
        ],
    )


@functools.cache
def _tc2(n, h, blk):
    grid = (n // blk,)

    def body(p_ref, dv_ref, ht1_ref, b1_ref, w2_ref, ht2_ref, hh2_ref):
        dv = dv_ref[...]
        agg = ((p_ref[0] + p_ref[1]) * dv
               + ht1_ref[...] * dv * dv + b1_ref[...])
        hrelu = jnp.maximum(agg, 0.0)
        ht2 = jnp.dot(hrelu, w2_ref[...], preferred_element_type=jnp.float32)
        ht2_ref[...] = ht2
        hh2_ref[...] = ht2 * dv

    return pl.pallas_call(
        body,
        grid=grid,
        in_specs=[
            pl.BlockSpec((NC, blk, 128), lambda i: (0, i, 0)),
            pl.BlockSpec((blk, 128), lambda i: (i, 0)),
            pl.BlockSpec((blk, 128), lambda i: (i, 0)),
            pl.BlockSpec((1, 128), lambda i: (0, 0)),
            pl.BlockSpec((h, h), lambda i: (0, 0)),
        ],
        out_specs=[
            pl.BlockSpec((blk, h), lambda i: (i, 0)),
            pl.BlockSpec((blk, h), lambda i: (i, 0)),
        ],
        out_shape=[
            jax.ShapeDtypeStruct((n, h), jnp.float32),
            jax.ShapeDtypeStruct((n, h), jnp.float32),
        ],
    )


@functools.cache
def _tc3(n, h, out_dim, g_graphs, blk):
    nblk = n // blk
    grid = (nblk,)

    def body(p_ref, dv_ref, ht2_ref, b2_ref, b_ref, wfc_ref, bfc_ref,
             out_ref, acc_ref, cnt_ref):
        i = pl.program_id(0)
        dv = dv_ref[...]
        agg = ((p_ref[0] + p_ref[1]) * dv
               + ht2_ref[...] * dv * dv + b2_ref[...])
        hrelu = jnp.maximum(agg, 0.0)
        bb = b_ref[0]
        gi = lax.broadcasted_iota(jnp.int32, (g_graphs, blk), 0)
        mt = (jnp.broadcast_to(bb, (g_graphs, blk)) == gi
              ).astype(jnp.float32)
        s_blk = jnp.dot(mt, hrelu, preferred_element_type=jnp.float32)
        c_blk = jnp.dot(mt, jnp.ones((blk, h), jnp.float32),
                        preferred_element_type=jnp.float32)

        @pl.when(i == 0)
        def _():
            acc_ref[...] = s_blk
            cnt_ref[...] = c_blk

        @pl.when(i > 0)
        def _():
            acc_ref[...] += s_blk
            cnt_ref[...] += c_blk

        @pl.when(i == nblk - 1)
        def _():
            pooled = acc_ref[...] / jnp.maximum(cnt_ref[...], 1.0)
            out_ref[...] = (jnp.dot(pooled, wfc_ref[...],
                                    preferred_element_type=jnp.float32)
                            + bfc_ref[...])

    return pl.pallas_call(
        body,
        grid=grid,
        in_specs=[
            pl.BlockSpec((NC, blk, 128), lambda i: (0, i, 0)),
            pl.BlockSpec((blk, 128), lambda i: (i, 0)),
            pl.BlockSpec((blk, 128), lambda i: (i, 0)),
            pl.BlockSpec((1, 128), lambda i: (0, 0)),
            pl.BlockSpec((1, 1, blk), lambda i: (i, 0, 0)),
            pl.BlockSpec((h, out_dim), lambda i: (0, 0)),
            pl.BlockSpec((1, out_dim), lambda i: (0, 0)),
        ],
        out_specs=pl.BlockSpec((g_graphs, out_dim), lambda i: (0, 0)),
        out_shape=jax.ShapeDtypeStruct((g_graphs, out_dim), jnp.float32),
        scratch_shapes=[
            pltpu.VMEM((g_graphs, h), jnp.float32),
            pltpu.VMEM((g_graphs, h), jnp.float32),
        ],
    )


def kernel(x, edge_index, edge_attr, batch, node_ids, emb_table,
           W1, b1, W2, b2, Wfc, bfc):
    n, d = x.shape
    e_edges = edge_attr.shape[0]
    ed = emb_table.shape[1]
    h = W2.shape[0]
    out_dim = Wfc.shape[1]
    g_graphs = 64
    blk = 1000

    src = edge_index[0]
    dst = edge_index[1]
    dst_r = dst.reshape(e_edges // RC, RC)
    w_r = edge_attr.reshape(e_edges // RC, RC)
    zeros = jnp.zeros((640, 128), jnp.float32)

    embed, deg2d = _sc_a(n, emb_table.shape[0], ed, e_edges)(
        node_ids, emb_table, dst_r, w_r)

    ht1, hhat1, dinv2d = _tc1(n, d, ed, h, blk)(
        x, embed, deg2d, W1[:d], W1[d:])

    part1 = _sc_b(n, e_edges)(hhat1, src, dst, edge_attr, zeros)

    ht2, hhat2 = _tc2(n, h, blk)(
        part1, dinv2d, ht1, b1.reshape(1, h), W2)

    part2 = _sc_b(n, e_edges)(hhat2, src, dst, edge_attr, zeros)

    out = _tc3(n, h, out_dim, g_graphs, blk)(
        part2, dinv2d, ht2, b2.reshape(1, h),
        batch.reshape(n // blk, 1, blk), Wfc, bfc.reshape(1, out_dim))
    return out
